# baseline (speedup 1.0000x reference)
import functools

import jax
import jax.numpy as jnp
import numpy as np
from jax import lax
from jax.experimental import pallas as pl
from jax.experimental.pallas import tpu as pltpu
from jax.experimental.pallas import tpu_sc as plsc

MAXLEN = 200
EMBED_DIM = 64
VOCAB = 1000000
NUM_SEQS = 4096
NUM_WORKERS = 32
BBLK = NUM_SEQS // NUM_WORKERS
LANES = 16
NREG = EMBED_DIM // LANES
TPITCH = 129


def _pos_encoding_np(maxlen, embed_dim):
    position = np.arange(maxlen)[:, np.newaxis]
    div_term = np.exp(np.arange(0, embed_dim, 2) * -(np.log(10000.0) / embed_dim))
    pos_enc = np.zeros((maxlen, embed_dim), dtype=np.float32)
    pos_enc[:, 0::2] = np.sin(position * div_term)
    pos_enc[:, 1::2] = np.cos(position * div_term)
    return pos_enc


_MESH = plsc.VectorSubcoreMesh(core_axis_name="c", subcore_axis_name="s")

JB = 512
NBLK = (VOCAB + 2 * JB - 1) // (2 * JB)
NPAIR = NBLK * JB


def _tr_body(lo_ref, hi_ref, eye_ref, out_ref):
    e = eye_ref[...]
    dn = (((0,), (0,)), ((), ()))
    out_ref[:, 0:EMBED_DIM] = lax.dot_general(
        lo_ref[...], e, dn, preferred_element_type=jnp.float32
    )
    out_ref[:, EMBED_DIM : 2 * EMBED_DIM] = lax.dot_general(
        hi_ref[...], e, dn, preferred_element_type=jnp.float32
    )


def _tc_linearize(tt, eye):
    return pl.pallas_call(
        _tr_body,
        out_shape=jax.ShapeDtypeStruct((NPAIR, 2 * EMBED_DIM), jnp.float32),
        grid=(NBLK,),
        in_specs=[
            pl.BlockSpec((EMBED_DIM, JB), lambda i: (0, 2 * i)),
            pl.BlockSpec((EMBED_DIM, JB), lambda i: (0, 2 * i + 1)),
            pl.BlockSpec((EMBED_DIM, EMBED_DIM), lambda i: (0, 0)),
        ],
        out_specs=pl.BlockSpec((JB, 2 * EMBED_DIM), lambda i: (i, 0)),
    )(tt, tt, eye)


_OUT_SHAPE = (MAXLEN, EMBED_DIM // 8, NUM_SEQS // 128, 8, 128)


@functools.partial(
    pl.kernel,
    out_type=jax.ShapeDtypeStruct(_OUT_SHAPE, jnp.float32),
    mesh=_MESH,
    scratch_types=[
        pltpu.VMEM((MAXLEN, BBLK), jnp.int32),
        pltpu.VMEM((2, BBLK), jnp.int32),
        pltpu.VMEM((2, BBLK, EMBED_DIM), jnp.float32),
        pltpu.VMEM((8, 8, TPITCH), jnp.float32),
        pltpu.VMEM((MAXLEN, EMBED_DIM), jnp.float32),
        pltpu.SemaphoreType.DMA,
    ],
    compiler_params=pltpu.CompilerParams(
        use_tc_tiling_on_sc=False, needs_layout_passes=False
    ),
)
def _emb_pos_kernel(xt_hbm, tab_hbm, pos_hbm, out_hbm, idx_v, jdx_v, rows_v, t_v,
                    pos_v, gsem):
    wid = lax.axis_index("s") * 2 + lax.axis_index("c")
    b0 = wid * BBLK

    pltpu.sync_copy(pos_hbm, pos_v)
    pltpu.sync_copy(xt_hbm.at[:, pl.ds(b0, BBLK)], idx_v)

    lane = lax.iota(jnp.int32, LANES)
    td = [(lane + 16 * j) // 8 for j in range(NREG)]
    ds_ = [lane % 8 for _ in range(NREG)]

    def fill_jdx(l, buf):
        for j in range(BBLK // LANES):
            sl = pl.ds(j * LANES, LANES)
            k = idx_v[l, sl]
            jdx_v[buf, sl] = (
                lax.shift_right_logical(k, 10) * (2 * JB)
                + (k & (JB - 1)) * 2
                + (lax.shift_right_logical(k, 9) & 1)
            )

    fill_jdx(0, 0)
    pltpu.async_copy(tab_hbm.at[jdx_v.at[0]], rows_v.at[0], gsem)

    def l_body(l, carry):
        cur = lax.rem(l, 2)
        nxt = lax.rem(l + 1, 2)

        @pl.when(l + 1 < MAXLEN)
        def _():
            fill_jdx(l + 1, nxt)
            pltpu.async_copy(tab_hbm.at[jdx_v.at[nxt]], rows_v.at[nxt], gsem)

        pltpu.make_async_copy(tab_hbm.at[jdx_v.at[cur]], rows_v.at[cur], gsem).wait()

        pv = tuple(pos_v[l, pl.ds(16 * j, LANES)] for j in range(NREG))

        def b_body(b, pvs):
            col = jnp.full((LANES,), b, dtype=jnp.int32)
            for j in range(NREG):
                v = rows_v[cur, b, pl.ds(16 * j, LANES)] + pvs[j]
                plsc.store_scatter(t_v, [td[j], ds_[j], col], v)
            return pvs

        lax.fori_loop(0, BBLK, b_body, pv, unroll=4)
        pltpu.sync_copy(t_v.at[:, :, pl.ds(0, 128)], out_hbm.at[l, :, wid])
        return carry

    lax.fori_loop(0, MAXLEN, l_body, 0)


def kernel(x, table):
    xt = x.T
    eye = jnp.eye(EMBED_DIM, dtype=jnp.float32)
    tab2 = _tc_linearize(table.T, eye)
    tab_rows = tab2.reshape(2 * NPAIR, EMBED_DIM)
    pos = jnp.asarray(_pos_encoding_np(MAXLEN, EMBED_DIM))
    k5 = _emb_pos_kernel(xt, tab_rows, pos)
    return k5.transpose(2, 4, 0, 1, 3).reshape(NUM_SEQS, MAXLEN, EMBED_DIM)


# speedup vs baseline: 1.7159x; 1.7159x over previous
import functools

import jax
import jax.numpy as jnp
import numpy as np
from jax import lax
from jax.experimental import pallas as pl
from jax.experimental.pallas import tpu as pltpu
from jax.experimental.pallas import tpu_sc as plsc

MAXLEN = 200
EMBED_DIM = 64
VOCAB = 1000000
NUM_SEQS = 4096
NUM_WORKERS = 32
BBLK = NUM_SEQS // NUM_WORKERS
LANES = 16
NREG = EMBED_DIM // LANES
TPITCH = 129


def _pos_encoding_np(maxlen, embed_dim):
    position = np.arange(maxlen)[:, np.newaxis]
    div_term = np.exp(np.arange(0, embed_dim, 2) * -(np.log(10000.0) / embed_dim))
    pos_enc = np.zeros((maxlen, embed_dim), dtype=np.float32)
    pos_enc[:, 0::2] = np.sin(position * div_term)
    pos_enc[:, 1::2] = np.cos(position * div_term)
    return pos_enc


_MESH = plsc.VectorSubcoreMesh(core_axis_name="c", subcore_axis_name="s")

JB = 8192
WBLK = 2 * JB
NBLK = (VOCAB + WBLK - 1) // WBLK
NPAIR = NBLK * JB
SUB = 512


def _tr_body(in_ref, eye_ref, out_ref):
    e = eye_ref[...]
    dn = (((0,), (0,)), ((), ()))
    for p in range(WBLK // SUB):
        a = in_ref[:, p * SUB : (p + 1) * SUB]
        at = lax.dot_general(a, e, dn, preferred_element_type=jnp.float32)
        r = (p * SUB) % JB
        c = (p * SUB) // JB * EMBED_DIM
        out_ref[r : r + SUB, c : c + EMBED_DIM] = at


def _tc_linearize(tt, eye):
    return pl.pallas_call(
        _tr_body,
        out_shape=jax.ShapeDtypeStruct((NPAIR, 2 * EMBED_DIM), jnp.float32),
        grid=(NBLK,),
        in_specs=[
            pl.BlockSpec((EMBED_DIM, WBLK), lambda i: (0, i)),
            pl.BlockSpec((EMBED_DIM, EMBED_DIM), lambda i: (0, 0)),
        ],
        out_specs=pl.BlockSpec((JB, 2 * EMBED_DIM), lambda i: (i, 0)),
    )(tt, eye)


_OUT_SHAPE = (MAXLEN, EMBED_DIM // 8, NUM_SEQS // 128, 8, 128)


@functools.partial(
    pl.kernel,
    out_type=jax.ShapeDtypeStruct(_OUT_SHAPE, jnp.float32),
    mesh=_MESH,
    scratch_types=[
        pltpu.VMEM((MAXLEN, BBLK), jnp.int32),
        pltpu.VMEM((2, BBLK), jnp.int32),
        pltpu.VMEM((2, BBLK, EMBED_DIM), jnp.float32),
        pltpu.VMEM((2, 8, 8, TPITCH), jnp.float32),
        pltpu.VMEM((MAXLEN, EMBED_DIM), jnp.float32),
        pltpu.SemaphoreType.DMA,
        pltpu.SemaphoreType.DMA,
    ],
    compiler_params=pltpu.CompilerParams(
        use_tc_tiling_on_sc=False,
        needs_layout_passes=False,
        disable_bounds_checks=True,
    ),
)
def _emb_pos_kernel(xt_hbm, tab_hbm, pos_hbm, out_hbm, idx_v, jdx_v, rows_v, t_v,
                    pos_v, gsem, wsem):
    wid = lax.axis_index("s") * 2 + lax.axis_index("c")
    b0 = wid * BBLK

    pltpu.sync_copy(pos_hbm, pos_v)
    pltpu.sync_copy(xt_hbm.at[:, pl.ds(b0, BBLK)], idx_v)

    lane = lax.iota(jnp.int32, LANES)
    td = [(lane + 16 * j) // 8 for j in range(NREG)]
    ds_ = [lane % 8 for _ in range(NREG)]

    def fill_jdx(l, buf):
        for j in range(BBLK // LANES):
            sl = pl.ds(j * LANES, LANES)
            k = idx_v[l, sl]
            jdx_v[buf, sl] = (
                lax.shift_right_logical(k, 14) * (2 * JB)
                + (k & (JB - 1)) * 2
                + (lax.shift_right_logical(k, 13) & 1)
            )

    fill_jdx(0, 0)
    pltpu.async_copy(tab_hbm.at[jdx_v.at[0]], rows_v.at[0], gsem)

    def l_body(l, carry):
        cur = lax.rem(l, 2)
        nxt = lax.rem(l + 1, 2)

        @pl.when(l + 1 < MAXLEN)
        def _():
            fill_jdx(l + 1, nxt)
            pltpu.async_copy(tab_hbm.at[jdx_v.at[nxt]], rows_v.at[nxt], gsem)

        pltpu.make_async_copy(tab_hbm.at[jdx_v.at[cur]], rows_v.at[cur], gsem).wait()

        @pl.when(l >= 2)
        def _():
            pltpu.make_async_copy(
                t_v.at[cur, :, :, pl.ds(0, 128)], out_hbm.at[l, :, wid], wsem
            ).wait()

        pv = tuple(pos_v[l, pl.ds(16 * j, LANES)] for j in range(NREG))

        def g_body(g, carry):
            gb = g * 8
            for i in range(8):
                b = gb + i
                col = jnp.full((LANES,), b, dtype=jnp.int32)
                for j in range(NREG):
                    v = rows_v[cur, b, pl.ds(16 * j, LANES)] + pv[j]
                    plsc.store_scatter(t_v.at[cur], [td[j], ds_[j], col], v)
            return carry

        lax.fori_loop(0, BBLK // 8, g_body, 0)
        pltpu.async_copy(
            t_v.at[cur, :, :, pl.ds(0, 128)], out_hbm.at[l, :, wid], wsem
        )
        return carry

    lax.fori_loop(0, MAXLEN, l_body, 0)
    for _ in range(2):
        pltpu.make_async_copy(
            t_v.at[0, :, :, pl.ds(0, 128)], out_hbm.at[0, :, wid], wsem
        ).wait()


def kernel(x, table):
    xt = x.T
    eye = jnp.eye(EMBED_DIM, dtype=jnp.float32)
    tab2 = _tc_linearize(table.T, eye)
    tab_rows = tab2.reshape(2 * NPAIR, EMBED_DIM)
    pos = jnp.asarray(_pos_encoding_np(MAXLEN, EMBED_DIM))
    k5 = _emb_pos_kernel(xt, tab_rows, pos)
    return k5.transpose(2, 4, 0, 1, 3).reshape(NUM_SEQS, MAXLEN, EMBED_DIM)


# speedup vs baseline: 3.1769x; 1.8514x over previous
import functools

import jax
import jax.numpy as jnp
import numpy as np
from jax import lax
from jax.experimental import pallas as pl
from jax.experimental.pallas import tpu as pltpu
from jax.experimental.pallas import tpu_sc as plsc

MAXLEN = 200
EMBED_DIM = 64
VOCAB = 1000000
NUM_SEQS = 4096
NUM_WORKERS = 32
BBLK = NUM_SEQS // NUM_WORKERS
LANES = 16
NREG = EMBED_DIM // LANES
TPITCH = 129


def _pos_encoding_np(maxlen, embed_dim):
    position = np.arange(maxlen)[:, np.newaxis]
    div_term = np.exp(np.arange(0, embed_dim, 2) * -(np.log(10000.0) / embed_dim))
    pos_enc = np.zeros((maxlen, embed_dim), dtype=np.float32)
    pos_enc[:, 0::2] = np.sin(position * div_term)
    pos_enc[:, 1::2] = np.cos(position * div_term)
    return pos_enc


_MESH = plsc.VectorSubcoreMesh(core_axis_name="c", subcore_axis_name="s")

JB = 8192
WBLK = 2 * JB
NBLK = (VOCAB + WBLK - 1) // WBLK
NPAIR = NBLK * JB
SUB = 512


def _tr_body(in_ref, eye_ref, out_ref):
    e = eye_ref[...]
    dn = (((0,), (0,)), ((), ()))
    for p in range(WBLK // SUB):
        a = in_ref[:, p * SUB : (p + 1) * SUB]
        at = lax.dot_general(a, e, dn, preferred_element_type=jnp.float32)
        r = (p * SUB) % JB
        c = (p * SUB) // JB * EMBED_DIM
        out_ref[r : r + SUB, c : c + EMBED_DIM] = at


def _tc_linearize(tt, eye):
    return pl.pallas_call(
        _tr_body,
        out_shape=jax.ShapeDtypeStruct((NPAIR, 2 * EMBED_DIM), jnp.float32),
        grid=(NBLK,),
        in_specs=[
            pl.BlockSpec((EMBED_DIM, WBLK), lambda i: (0, i)),
            pl.BlockSpec((EMBED_DIM, EMBED_DIM), lambda i: (0, 0)),
        ],
        out_specs=pl.BlockSpec((JB, 2 * EMBED_DIM), lambda i: (i, 0)),
    )(tt, eye)


_OUT_SHAPE = (MAXLEN, EMBED_DIM // 8, NUM_SEQS // 128, 8, 128)


@functools.partial(
    pl.kernel,
    out_type=jax.ShapeDtypeStruct(_OUT_SHAPE, jnp.float32),
    mesh=_MESH,
    scratch_types=[
        pltpu.VMEM((MAXLEN, BBLK), jnp.int32),
        pltpu.VMEM((2, BBLK), jnp.int32),
        pltpu.VMEM((2, BBLK, EMBED_DIM), jnp.float32),
        pltpu.VMEM((2, 8, 8, TPITCH), jnp.float32),
        pltpu.VMEM((MAXLEN, EMBED_DIM), jnp.float32),
        pltpu.SemaphoreType.DMA,
        pltpu.SemaphoreType.DMA,
    ],
    compiler_params=pltpu.CompilerParams(
        use_tc_tiling_on_sc=False,
        needs_layout_passes=False,
        disable_bounds_checks=True,
    ),
)
def _emb_pos_kernel(xt_hbm, tab_hbm, pos_hbm, out_hbm, idx_v, jdx_v, rows_v, t_v,
                    pos_v, gsem, wsem):
    wid = lax.axis_index("s") * 2 + lax.axis_index("c")
    b0 = wid * BBLK

    pltpu.sync_copy(pos_hbm, pos_v)
    pltpu.sync_copy(xt_hbm.at[:, pl.ds(b0, BBLK)], idx_v)

    lane = lax.iota(jnp.int32, LANES)
    td = [(lane + 16 * j) // 8 for j in range(NREG)]
    ds_ = [lane % 8 for _ in range(NREG)]

    def fill_jdx(l, buf):
        for j in range(BBLK // LANES):
            sl = pl.ds(j * LANES, LANES)
            k = idx_v[l, sl]
            jdx_v[buf, sl] = (
                lax.shift_right_logical(k, 14) * (2 * JB)
                + (k & (JB - 1)) * 2
                + (lax.shift_right_logical(k, 13) & 1)
            )

    fill_jdx(0, 0)
    pltpu.async_copy(tab_hbm.at[jdx_v.at[0]], rows_v.at[0], gsem)

    def l_body(l, carry):
        cur = lax.rem(l, 2)
        nxt = lax.rem(l + 1, 2)

        @pl.when(l + 1 < MAXLEN)
        def _():
            fill_jdx(l + 1, nxt)
            pltpu.async_copy(tab_hbm.at[jdx_v.at[nxt]], rows_v.at[nxt], gsem)

        pltpu.make_async_copy(tab_hbm.at[jdx_v.at[cur]], rows_v.at[cur], gsem).wait()

        @pl.when(l >= 2)
        def _():
            pltpu.make_async_copy(
                t_v.at[cur, :, :, pl.ds(0, 128)], out_hbm.at[l, :, wid], wsem
            ).wait()

        pv = tuple(pos_v[l, pl.ds(16 * j, LANES)] for j in range(NREG))

        @functools.partial(plsc.parallel_loop, 0, BBLK, unroll=4)
        def _(b):
            col = jnp.full((LANES,), b, dtype=jnp.int32)
            vs = [rows_v[cur, b, pl.ds(16 * j, LANES)] for j in range(NREG)]
            vs = [vs[j] + pv[j] for j in range(NREG)]
            for j in range(NREG):
                plsc.store_scatter(t_v.at[cur], [td[j], ds_[j], col], vs[j])
        pltpu.async_copy(
            t_v.at[cur, :, :, pl.ds(0, 128)], out_hbm.at[l, :, wid], wsem
        )
        return carry

    lax.fori_loop(0, MAXLEN, l_body, 0)
    for _ in range(2):
        pltpu.make_async_copy(
            t_v.at[0, :, :, pl.ds(0, 128)], out_hbm.at[0, :, wid], wsem
        ).wait()


def kernel(x, table):
    xt = x.T
    eye = jnp.eye(EMBED_DIM, dtype=jnp.float32)
    tab2 = _tc_linearize(table.T, eye)
    tab_rows = tab2.reshape(2 * NPAIR, EMBED_DIM)
    pos = jnp.asarray(_pos_encoding_np(MAXLEN, EMBED_DIM))
    k5 = _emb_pos_kernel(xt, tab_rows, pos)
    return k5.transpose(2, 4, 0, 1, 3).reshape(NUM_SEQS, MAXLEN, EMBED_DIM)
